# baseline (speedup 1.0000x reference)
import functools

import jax
import jax.numpy as jnp
from jax import lax
from jax.experimental import pallas as pl
from jax.experimental.pallas import tpu as pltpu
from jax.experimental.pallas import tpu_sc as plsc

NC, NS = 2, 16
NW = NC * NS
CHUNK = 128
DEGW = 8
DEPTH = 8
SP_PAD = 10240


def _sc_mesh():
    return plsc.VectorSubcoreMesh(
        core_axis_name="c", subcore_axis_name="s", num_cores=NC, num_subcores=NS
    )


def _make_deg_kernel(n, nchunk):
    out_rows = n // NS
    zrows = SP_PAD // NS

    @functools.partial(
        pl.kernel,
        out_type=jax.ShapeDtypeStruct((NC * n, DEGW), jnp.float32),
        mesh=_sc_mesh(),
        scratch_types=[
            pltpu.VMEM((nchunk, CHUNK), jnp.int32),
            pltpu.VMEM((CHUNK, DEGW), jnp.float32),
            pltpu.VMEM((out_rows, DEGW), jnp.float32),
            pltpu.VMEM_SHARED((SP_PAD, DEGW), jnp.float32),
        ]
        + [pltpu.SemaphoreType.DMA] * DEPTH,
        compiler_params=pltpu.CompilerParams(use_tc_tiling_on_sc=False),
    )
    def deg_kernel(dst_hbm, ones_hbm, deg_out, dst_v, ones_v, bounce_v,
                   deg_sp, *sems):
        cid = lax.axis_index("c")
        sid = lax.axis_index("s")
        wid = sid * NC + cid

        pltpu.sync_copy(ones_hbm.at[0], ones_v)
        pltpu.sync_copy(ones_hbm.at[1], bounce_v.at[pl.ds(0, CHUNK)])

        zsrc = bounce_v.at[pl.ds(0, CHUNK)]
        for r in range(zrows // CHUNK):
            pltpu.sync_copy(
                zsrc, deg_sp.at[pl.ds(sid * zrows + r * CHUNK, CHUNK)]
            )
        pltpu.sync_copy(dst_hbm.at[wid], dst_v)
        plsc.subcore_barrier()

        def body(t, _):
            for k in range(DEPTH):
                j = DEPTH * t + k
                pltpu.async_copy(
                    ones_v, deg_sp.at[dst_v.at[j]], sems[k], add=True
                )
            for k in range(DEPTH):
                j = DEPTH * t + k
                pltpu.make_async_copy(
                    ones_v, deg_sp.at[dst_v.at[j]], sems[k]
                ).wait()
            return 0

        lax.fori_loop(0, nchunk // DEPTH, body, 0)
        plsc.subcore_barrier()

        sl = pl.ds(sid * out_rows, out_rows)
        pltpu.sync_copy(deg_sp.at[sl], bounce_v)
        pltpu.sync_copy(
            bounce_v, deg_out.at[pl.ds(cid * n + sid * out_rows, out_rows)]
        )

    return deg_kernel


def _make_agg_kernel(n, nchunk, dc):
    out_rows = n // NS
    zrows = SP_PAD // NS

    @functools.partial(
        pl.kernel,
        out_type=jax.ShapeDtypeStruct((NC * n, dc), jnp.float32),
        mesh=_sc_mesh(),
        scratch_types=[
            pltpu.VMEM((nchunk, CHUNK), jnp.int32),
            pltpu.VMEM((nchunk, CHUNK), jnp.int32),
            pltpu.VMEM((DEPTH, CHUNK, dc), jnp.float32),
            pltpu.VMEM((out_rows, dc), jnp.float32),
            pltpu.VMEM_SHARED((SP_PAD, dc), jnp.float32),
            pltpu.VMEM_SHARED((n, dc), jnp.float32),
        ]
        + [pltpu.SemaphoreType.DMA] * DEPTH,
        compiler_params=pltpu.CompilerParams(use_tc_tiling_on_sc=False),
    )
    def agg_kernel(src_hbm, dst_hbm, g_hbm, s_out, src_v, dst_v, bufs,
                   bounce_v, s_sp, g_sp, *sems):
        cid = lax.axis_index("c")
        sid = lax.axis_index("s")
        wid = sid * NC + cid

        gsl = pl.ds(sid * out_rows, out_rows)
        pltpu.sync_copy(g_hbm.at[gsl], bounce_v)
        pltpu.sync_copy(bounce_v, g_sp.at[gsl])

        zeros16 = jnp.zeros((16,), jnp.float32)

        def fill(i, _):
            for c0 in range(0, dc, 16):
                bounce_v[i, pl.ds(c0, 16)] = zeros16
            return 0

        lax.fori_loop(0, CHUNK, fill, 0)

        zsrc = bounce_v.at[pl.ds(0, CHUNK)]
        for r in range(zrows // CHUNK):
            pltpu.sync_copy(zsrc, s_sp.at[pl.ds(sid * zrows + r * CHUNK, CHUNK)])
        pltpu.sync_copy(src_hbm.at[wid], src_v)
        pltpu.sync_copy(dst_hbm.at[wid], dst_v)
        plsc.subcore_barrier()

        for k in range(DEPTH):
            pltpu.async_copy(g_sp.at[src_v.at[k]], bufs.at[k], sems[k])

        def body(t, _):
            for k in range(DEPTH):
                j = DEPTH * t + k
                pltpu.make_async_copy(
                    g_sp.at[src_v.at[j]], bufs.at[k], sems[k]
                ).wait()
                pltpu.sync_copy(bufs.at[k], s_sp.at[dst_v.at[j]], add=True)

                @pl.when(j + DEPTH < nchunk)
                def _():
                    pltpu.async_copy(
                        g_sp.at[src_v.at[j + DEPTH]], bufs.at[k], sems[k]
                    )

            return 0

        lax.fori_loop(0, nchunk // DEPTH, body, 0)
        plsc.subcore_barrier()

        sl = pl.ds(sid * out_rows, out_rows)
        pltpu.sync_copy(s_sp.at[sl], bounce_v)
        pltpu.sync_copy(
            bounce_v, s_out.at[pl.ds(cid * n + sid * out_rows, out_rows)]
        )

    return agg_kernel


def _rsqrt_sc(d):
    i = plsc.bitcast(d, jnp.int32)
    y = plsc.bitcast(
        jnp.full((16,), 0x5F3759DF, jnp.int32) - lax.shift_right_logical(i, 1),
        jnp.float32,
    )
    for _ in range(3):
        y = y * (1.5 - 0.5 * d * y * y)
    return y


def _make_fin_kernel(n, dc, dout):
    rows_a = 320
    rows_l = n - (NW - 1) * rows_a

    @functools.partial(
        pl.kernel,
        out_type=(
            jax.ShapeDtypeStruct((n, dout), jnp.float32),
            jax.ShapeDtypeStruct((n, dout), jnp.float32),
        ),
        mesh=_sc_mesh(),
        scratch_types=[
            pltpu.VMEM((rows_a, dc), jnp.float32),
            pltpu.VMEM((rows_a, dc), jnp.float32),
            pltpu.VMEM((rows_a, dc), jnp.float32),
            pltpu.VMEM((rows_a, DEGW), jnp.float32),
            pltpu.VMEM((rows_a, DEGW), jnp.float32),
            pltpu.VMEM((rows_a, dout), jnp.float32),
            pltpu.VMEM((rows_a, dout), jnp.float32),
            pltpu.VMEM((2, 16), jnp.float32),
            pltpu.VMEM((16,), jnp.float32),
        ],
        compiler_params=pltpu.CompilerParams(
            use_tc_tiling_on_sc=False, needs_layout_passes=False
        ),
    )
    def fin_kernel(s_hbm, g_hbm, deg_hbm, b_hbm, mu_out, lo_out,
                   s0_v, s1_v, g_v, d0_v, d1_v, mu_v, lo_v, b_v, dinv_v):
        cid = lax.axis_index("c")
        sid = lax.axis_index("s")
        wid = sid * NC + cid
        base = wid * rows_a

        pltpu.sync_copy(b_hbm, b_v)

        def run(rows):
            rsl = pl.ds(0, rows)
            pltpu.sync_copy(s_hbm.at[pl.ds(base, rows)], s0_v.at[rsl])
            pltpu.sync_copy(s_hbm.at[pl.ds(n + base, rows)], s1_v.at[rsl])
            pltpu.sync_copy(g_hbm.at[pl.ds(base, rows)], g_v.at[rsl])
            pltpu.sync_copy(deg_hbm.at[pl.ds(base, rows)], d0_v.at[rsl])
            pltpu.sync_copy(deg_hbm.at[pl.ds(n + base, rows)], d1_v.at[rsl])

            bm = b_v[0]
            bl = b_v[1]
            zc = jnp.zeros((16,), jnp.int32)

            def group(t, _):
                rbase = t * 16
                for i in range(16):
                    r = rbase + i
                    ridx = jnp.full((16,), r, jnp.int32)
                    dr0 = plsc.load_gather(d0_v, [ridx, zc])
                    dr1 = plsc.load_gather(d1_v, [ridx, zc])
                    sv = _rsqrt_sc(dr0 + dr1 + 1.0)
                    lo16 = pl.ds(0, 16)
                    hi16 = pl.ds(16, 16)
                    mu_v[r] = (
                        s0_v[r, lo16] + s1_v[r, lo16] + g_v[r, lo16]
                    ) * sv + bm
                    lo_v[r] = (
                        s0_v[r, hi16] + s1_v[r, hi16] + g_v[r, hi16]
                    ) * sv + bl
                return 0

            lax.fori_loop(0, rows // 16, group, 0)
            pltpu.sync_copy(mu_v.at[rsl], mu_out.at[pl.ds(base, rows)])
            pltpu.sync_copy(lo_v.at[rsl], lo_out.at[pl.ds(base, rows)])

        @pl.when(wid < NW - 1)
        def _():
            run(rows_a)

        @pl.when(wid == NW - 1)
        def _():
            run(rows_l)

    return fin_kernel


def _matmul_stage(x, wcat, n, dc, block):
    def body(x_ref, w_ref, h_ref):
        h_ref[...] = jnp.dot(
            x_ref[...], w_ref[...], preferred_element_type=jnp.float32
        )

    return pl.pallas_call(
        body,
        grid=(n // block,),
        in_specs=[
            pl.BlockSpec((block, x.shape[1]), lambda i: (i, 0)),
            pl.BlockSpec((wcat.shape[0], dc), lambda i: (0, 0)),
        ],
        out_specs=pl.BlockSpec((block, dc), lambda i: (i, 0)),
        out_shape=jax.ShapeDtypeStruct((n, dc), jnp.float32),
    )(x, wcat)


def _scale_stage(h, deg, n, dc):
    def body(h_ref, d0_ref, d1_ref, g_ref):
        degsum = d0_ref[:, 0] + d1_ref[:, 0] + 1.0
        g_ref[...] = h_ref[...] * lax.rsqrt(degsum)[:, None]

    return pl.pallas_call(
        body,
        grid=(1,),
        in_specs=[
            pl.BlockSpec((n, dc), lambda i: (0, 0)),
            pl.BlockSpec((n, DEGW), lambda i: (0, 0)),
            pl.BlockSpec((n, DEGW), lambda i: (1, 0)),
        ],
        out_specs=pl.BlockSpec((n, dc), lambda i: (0, 0)),
        out_shape=jax.ShapeDtypeStruct((n, dc), jnp.float32),
    )(h, deg, deg)


def _finalize_stage(s_parts, g, deg, b_mu, b_logstd, n, dc, dout):
    def body(s0_ref, s1_ref, g_ref, d0_ref, d1_ref, bm_ref, bl_ref,
             mu_ref, lo_ref):
        degsum = d0_ref[:, 0] + d1_ref[:, 0] + 1.0
        dinv = lax.rsqrt(degsum)
        out = (s0_ref[...] + s1_ref[...] + g_ref[...]) * dinv[:, None]
        mu_ref[...] = out[:, :dout] + bm_ref[...]
        lo_ref[...] = out[:, dout:] + bl_ref[...]

    return pl.pallas_call(
        body,
        grid=(1,),
        in_specs=[
            pl.BlockSpec((n, dc), lambda i: (0, 0)),
            pl.BlockSpec((n, dc), lambda i: (1, 0)),
            pl.BlockSpec((n, dc), lambda i: (0, 0)),
            pl.BlockSpec((n, DEGW), lambda i: (0, 0)),
            pl.BlockSpec((n, DEGW), lambda i: (1, 0)),
            pl.BlockSpec((1, dout), lambda i: (0, 0)),
            pl.BlockSpec((1, dout), lambda i: (0, 0)),
        ],
        out_specs=[
            pl.BlockSpec((n, dout), lambda i: (0, 0)),
            pl.BlockSpec((n, dout), lambda i: (0, 0)),
        ],
        out_shape=[
            jax.ShapeDtypeStruct((n, dout), jnp.float32),
            jax.ShapeDtypeStruct((n, dout), jnp.float32),
        ],
    )(s_parts, s_parts, g, deg, deg, b_mu, b_logstd)


@jax.jit
def kernel(x, W_mu, b_mu, W_logstd, b_logstd, edge_index):
    n, din = x.shape
    dout = W_mu.shape[1]
    dc = 2 * dout
    e = edge_index.shape[1]
    block = 1000

    e_pad = ((e + NW * CHUNK - 1) // (NW * CHUNK)) * (NW * CHUNK)
    nchunk = e_pad // (NW * CHUNK)

    src = edge_index[0].astype(jnp.int32)
    dst = edge_index[1].astype(jnp.int32)
    src = jnp.concatenate([src, jnp.zeros((e_pad - e,), jnp.int32)])
    dst = jnp.concatenate([dst, jnp.full((e_pad - e,), SP_PAD - 1, jnp.int32)])
    src = src.reshape(NW, nchunk, CHUNK)
    dst = dst.reshape(NW, nchunk, CHUNK)

    wcat = jnp.concatenate([W_mu, W_logstd], axis=1)

    oz = jnp.stack(
        [jnp.ones((CHUNK, DEGW), jnp.float32), jnp.zeros((CHUNK, DEGW), jnp.float32)]
    )
    h = _matmul_stage(x, wcat, n, dc, block)
    deg = _make_deg_kernel(n, nchunk)(dst, oz)
    g = _scale_stage(h, deg, n, dc)
    s_parts = _make_agg_kernel(n, nchunk, dc)(src, dst, g)
    bcat = jnp.stack([b_mu, b_logstd])
    return _make_fin_kernel(n, dc, dout)(s_parts, g, deg, bcat)


# speedup vs baseline: 1.0478x; 1.0478x over previous
import functools

import jax
import jax.numpy as jnp
from jax import lax
from jax.experimental import pallas as pl
from jax.experimental.pallas import tpu as pltpu
from jax.experimental.pallas import tpu_sc as plsc

NC, NS = 2, 16
NW = NC * NS
CHUNK = 128
DEGW = 8
DEPTH = 10
SP_PAD = 10240


def _sc_mesh():
    return plsc.VectorSubcoreMesh(
        core_axis_name="c", subcore_axis_name="s", num_cores=NC, num_subcores=NS
    )


def _make_deg_kernel(n, nchunk):
    out_rows = n // NS
    zrows = SP_PAD // NS

    @functools.partial(
        pl.kernel,
        out_type=jax.ShapeDtypeStruct((NC * n, DEGW), jnp.float32),
        mesh=_sc_mesh(),
        scratch_types=[
            pltpu.VMEM((nchunk, CHUNK), jnp.int32),
            pltpu.VMEM((CHUNK, DEGW), jnp.float32),
            pltpu.VMEM((out_rows, DEGW), jnp.float32),
            pltpu.VMEM_SHARED((SP_PAD, DEGW), jnp.float32),
        ]
        + [pltpu.SemaphoreType.DMA] * DEPTH,
        compiler_params=pltpu.CompilerParams(use_tc_tiling_on_sc=False),
    )
    def deg_kernel(dst_hbm, ones_hbm, deg_out, dst_v, ones_v, bounce_v,
                   deg_sp, *sems):
        cid = lax.axis_index("c")
        sid = lax.axis_index("s")
        wid = sid * NC + cid

        pltpu.sync_copy(ones_hbm.at[0], ones_v)
        pltpu.sync_copy(ones_hbm.at[1], bounce_v.at[pl.ds(0, CHUNK)])

        zsrc = bounce_v.at[pl.ds(0, CHUNK)]
        for r in range(zrows // CHUNK):
            pltpu.sync_copy(
                zsrc, deg_sp.at[pl.ds(sid * zrows + r * CHUNK, CHUNK)]
            )
        pltpu.sync_copy(dst_hbm.at[wid], dst_v)
        plsc.subcore_barrier()

        def body(t, _):
            for k in range(DEPTH):
                j = DEPTH * t + k
                pltpu.async_copy(
                    ones_v, deg_sp.at[dst_v.at[j]], sems[k], add=True
                )
            for k in range(DEPTH):
                j = DEPTH * t + k
                pltpu.make_async_copy(
                    ones_v, deg_sp.at[dst_v.at[j]], sems[k]
                ).wait()
            return 0

        lax.fori_loop(0, nchunk // DEPTH, body, 0)
        plsc.subcore_barrier()

        sl = pl.ds(sid * out_rows, out_rows)
        pltpu.sync_copy(deg_sp.at[sl], bounce_v)
        pltpu.sync_copy(
            bounce_v, deg_out.at[pl.ds(cid * n + sid * out_rows, out_rows)]
        )

    return deg_kernel


def _make_agg_kernel(n, nchunk, dc):
    out_rows = n // NS
    zrows = SP_PAD // NS

    @functools.partial(
        pl.kernel,
        out_type=jax.ShapeDtypeStruct((NC * n, dc), jnp.float32),
        mesh=_sc_mesh(),
        scratch_types=[
            pltpu.VMEM((nchunk, CHUNK), jnp.int32),
            pltpu.VMEM((nchunk, CHUNK), jnp.int32),
            pltpu.VMEM((DEPTH, CHUNK, dc), jnp.float32),
            pltpu.VMEM((out_rows, dc), jnp.float32),
            pltpu.VMEM_SHARED((SP_PAD, dc), jnp.float32),
            pltpu.VMEM_SHARED((n, dc), jnp.float32),
        ]
        + [pltpu.SemaphoreType.DMA] * DEPTH,
        compiler_params=pltpu.CompilerParams(use_tc_tiling_on_sc=False),
    )
    def agg_kernel(src_hbm, dst_hbm, g_hbm, s_out, src_v, dst_v, bufs,
                   bounce_v, s_sp, g_sp, *sems):
        cid = lax.axis_index("c")
        sid = lax.axis_index("s")
        wid = sid * NC + cid

        gsl = pl.ds(sid * out_rows, out_rows)
        pltpu.sync_copy(g_hbm.at[gsl], bounce_v)
        pltpu.sync_copy(bounce_v, g_sp.at[gsl])

        zeros16 = jnp.zeros((16,), jnp.float32)

        def fill(i, _):
            for c0 in range(0, dc, 16):
                bounce_v[i, pl.ds(c0, 16)] = zeros16
            return 0

        lax.fori_loop(0, CHUNK, fill, 0)

        zsrc = bounce_v.at[pl.ds(0, CHUNK)]
        for r in range(zrows // CHUNK):
            pltpu.sync_copy(zsrc, s_sp.at[pl.ds(sid * zrows + r * CHUNK, CHUNK)])
        pltpu.sync_copy(src_hbm.at[wid], src_v)
        pltpu.sync_copy(dst_hbm.at[wid], dst_v)
        plsc.subcore_barrier()

        for k in range(DEPTH):
            pltpu.async_copy(g_sp.at[src_v.at[k]], bufs.at[k], sems[k])

        def body(t, _):
            for k in range(DEPTH):
                j = DEPTH * t + k
                pltpu.make_async_copy(
                    g_sp.at[src_v.at[j]], bufs.at[k], sems[k]
                ).wait()
                pltpu.sync_copy(bufs.at[k], s_sp.at[dst_v.at[j]], add=True)

                @pl.when(j + DEPTH < nchunk)
                def _():
                    pltpu.async_copy(
                        g_sp.at[src_v.at[j + DEPTH]], bufs.at[k], sems[k]
                    )

            return 0

        lax.fori_loop(0, nchunk // DEPTH, body, 0)
        plsc.subcore_barrier()

        sl = pl.ds(sid * out_rows, out_rows)
        pltpu.sync_copy(s_sp.at[sl], bounce_v)
        pltpu.sync_copy(
            bounce_v, s_out.at[pl.ds(cid * n + sid * out_rows, out_rows)]
        )

    return agg_kernel


def _matmul_stage(x, wcat, n, dc, block):
    def body(x_ref, w_ref, h_ref):
        h_ref[...] = jnp.dot(
            x_ref[...], w_ref[...], preferred_element_type=jnp.float32
        )

    return pl.pallas_call(
        body,
        grid=(n // block,),
        in_specs=[
            pl.BlockSpec((block, x.shape[1]), lambda i: (i, 0)),
            pl.BlockSpec((wcat.shape[0], dc), lambda i: (0, 0)),
        ],
        out_specs=pl.BlockSpec((block, dc), lambda i: (i, 0)),
        out_shape=jax.ShapeDtypeStruct((n, dc), jnp.float32),
    )(x, wcat)


def _scale_stage(h, deg, n, dc):
    def body(h_ref, d0_ref, d1_ref, g_ref):
        degsum = d0_ref[:, 0] + d1_ref[:, 0] + 1.0
        g_ref[...] = h_ref[...] * lax.rsqrt(degsum)[:, None]

    return pl.pallas_call(
        body,
        grid=(1,),
        in_specs=[
            pl.BlockSpec((n, dc), lambda i: (0, 0)),
            pl.BlockSpec((n, DEGW), lambda i: (0, 0)),
            pl.BlockSpec((n, DEGW), lambda i: (1, 0)),
        ],
        out_specs=pl.BlockSpec((n, dc), lambda i: (0, 0)),
        out_shape=jax.ShapeDtypeStruct((n, dc), jnp.float32),
    )(h, deg, deg)


def _finalize_stage(s_parts, g, deg, b_mu, b_logstd, n, dc, dout):
    def body(s0_ref, s1_ref, g_ref, d0_ref, d1_ref, bm_ref, bl_ref,
             mu_ref, lo_ref):
        degsum = d0_ref[:, 0] + d1_ref[:, 0] + 1.0
        dinv = lax.rsqrt(degsum)
        out = (s0_ref[...] + s1_ref[...] + g_ref[...]) * dinv[:, None]
        mu_ref[...] = out[:, :dout] + bm_ref[...]
        lo_ref[...] = out[:, dout:] + bl_ref[...]

    return pl.pallas_call(
        body,
        grid=(1,),
        in_specs=[
            pl.BlockSpec((n, dc), lambda i: (0, 0)),
            pl.BlockSpec((n, dc), lambda i: (1, 0)),
            pl.BlockSpec((n, dc), lambda i: (0, 0)),
            pl.BlockSpec((n, DEGW), lambda i: (0, 0)),
            pl.BlockSpec((n, DEGW), lambda i: (1, 0)),
            pl.BlockSpec((1, dout), lambda i: (0, 0)),
            pl.BlockSpec((1, dout), lambda i: (0, 0)),
        ],
        out_specs=[
            pl.BlockSpec((n, dout), lambda i: (0, 0)),
            pl.BlockSpec((n, dout), lambda i: (0, 0)),
        ],
        out_shape=[
            jax.ShapeDtypeStruct((n, dout), jnp.float32),
            jax.ShapeDtypeStruct((n, dout), jnp.float32),
        ],
    )(s_parts, s_parts, g, deg, deg, b_mu, b_logstd)


@jax.jit
def kernel(x, W_mu, b_mu, W_logstd, b_logstd, edge_index):
    n, din = x.shape
    dout = W_mu.shape[1]
    dc = 2 * dout
    e = edge_index.shape[1]
    block = 1000

    e_pad = ((e + NW * CHUNK - 1) // (NW * CHUNK)) * (NW * CHUNK)
    nchunk = e_pad // (NW * CHUNK)

    src = edge_index[0].astype(jnp.int32)
    dst = edge_index[1].astype(jnp.int32)
    src = jnp.concatenate([src, jnp.zeros((e_pad - e,), jnp.int32)])
    dst = jnp.concatenate([dst, jnp.full((e_pad - e,), SP_PAD - 1, jnp.int32)])
    src = src.reshape(NW, nchunk, CHUNK)
    dst = dst.reshape(NW, nchunk, CHUNK)

    wcat = jnp.concatenate([W_mu, W_logstd], axis=1)

    oz = jnp.stack(
        [jnp.ones((CHUNK, DEGW), jnp.float32), jnp.zeros((CHUNK, DEGW), jnp.float32)]
    )
    h = _matmul_stage(x, wcat, n, dc, block)
    deg = _make_deg_kernel(n, nchunk)(dst, oz)
    g = _scale_stage(h, deg, n, dc)
    s_parts = _make_agg_kernel(n, nchunk, dc)(src, dst, g)
    return _finalize_stage(
        s_parts, g, deg, b_mu.reshape(1, dout), b_logstd.reshape(1, dout),
        n, dc, dout
    )


# speedup vs baseline: 1.1047x; 1.0543x over previous
import functools

import jax
import jax.numpy as jnp
from jax import lax
from jax.experimental import pallas as pl
from jax.experimental.pallas import tpu as pltpu
from jax.experimental.pallas import tpu_sc as plsc

NC, NS = 2, 16
NW = NC * NS
CHUNK = 128
DEGW = 8
DEPTH = 8
SP_PAD = 10240


def _sc_mesh():
    return plsc.VectorSubcoreMesh(
        core_axis_name="c", subcore_axis_name="s", num_cores=NC, num_subcores=NS
    )


def _make_deg_kernel(n, nchunk):
    out_rows = n // NS
    zrows = SP_PAD // NS

    @functools.partial(
        pl.kernel,
        out_type=jax.ShapeDtypeStruct((NC * n, DEGW), jnp.float32),
        mesh=_sc_mesh(),
        scratch_types=[
            pltpu.VMEM((nchunk, CHUNK), jnp.int32),
            pltpu.VMEM((CHUNK, DEGW), jnp.float32),
            pltpu.VMEM((out_rows, DEGW), jnp.float32),
            pltpu.VMEM_SHARED((SP_PAD, DEGW), jnp.float32),
        ]
        + [pltpu.SemaphoreType.DMA] * DEPTH,
        compiler_params=pltpu.CompilerParams(use_tc_tiling_on_sc=False),
    )
    def deg_kernel(dst_hbm, ones_hbm, deg_out, dst_v, ones_v, bounce_v,
                   deg_sp, *sems):
        cid = lax.axis_index("c")
        sid = lax.axis_index("s")
        wid = sid * NC + cid

        pltpu.sync_copy(ones_hbm.at[0], ones_v)
        pltpu.sync_copy(ones_hbm.at[1], bounce_v.at[pl.ds(0, CHUNK)])

        zsrc = bounce_v.at[pl.ds(0, CHUNK)]
        for r in range(zrows // CHUNK):
            pltpu.sync_copy(
                zsrc, deg_sp.at[pl.ds(sid * zrows + r * CHUNK, CHUNK)]
            )
        pltpu.sync_copy(dst_hbm.at[wid], dst_v)
        plsc.subcore_barrier()

        def body(t, _):
            for k in range(DEPTH):
                j = DEPTH * t + k
                pltpu.async_copy(
                    ones_v, deg_sp.at[dst_v.at[j]], sems[k], add=True
                )
            for k in range(DEPTH):
                j = DEPTH * t + k
                pltpu.make_async_copy(
                    ones_v, deg_sp.at[dst_v.at[j]], sems[k]
                ).wait()
            return 0

        lax.fori_loop(0, nchunk // DEPTH, body, 0)
        plsc.subcore_barrier()

        sl = pl.ds(sid * out_rows, out_rows)
        pltpu.sync_copy(deg_sp.at[sl], bounce_v)
        pltpu.sync_copy(
            bounce_v, deg_out.at[pl.ds(cid * n + sid * out_rows, out_rows)]
        )

    return deg_kernel


def _make_agg_kernel(n, nchunk, dc):
    out_rows = n // NS
    zrows = SP_PAD // NS

    @functools.partial(
        pl.kernel,
        out_type=jax.ShapeDtypeStruct((NC * n, dc), jnp.float32),
        mesh=_sc_mesh(),
        scratch_types=[
            pltpu.VMEM((nchunk, CHUNK), jnp.int32),
            pltpu.VMEM((nchunk, CHUNK), jnp.int32),
            pltpu.VMEM((DEPTH, CHUNK, dc), jnp.float32),
            pltpu.VMEM((out_rows, dc), jnp.float32),
            pltpu.VMEM_SHARED((SP_PAD, dc), jnp.float32),
            pltpu.VMEM_SHARED((n, dc), jnp.float32),
        ]
        + [pltpu.SemaphoreType.DMA] * DEPTH,
        compiler_params=pltpu.CompilerParams(use_tc_tiling_on_sc=False),
    )
    def agg_kernel(src_hbm, dst_hbm, g_hbm, s_out, src_v, dst_v, bufs,
                   bounce_v, s_sp, g_sp, *sems):
        cid = lax.axis_index("c")
        sid = lax.axis_index("s")
        wid = sid * NC + cid

        gsl = pl.ds(sid * out_rows, out_rows)
        pltpu.sync_copy(g_hbm.at[gsl], bounce_v)
        pltpu.sync_copy(bounce_v, g_sp.at[gsl])

        zeros16 = jnp.zeros((16,), jnp.float32)

        def fill(i, _):
            for c0 in range(0, dc, 16):
                bounce_v[i, pl.ds(c0, 16)] = zeros16
            return 0

        lax.fori_loop(0, CHUNK, fill, 0)

        zsrc = bounce_v.at[pl.ds(0, CHUNK)]
        for r in range(zrows // CHUNK):
            pltpu.sync_copy(zsrc, s_sp.at[pl.ds(sid * zrows + r * CHUNK, CHUNK)])
        pltpu.sync_copy(src_hbm.at[wid], src_v)
        pltpu.sync_copy(dst_hbm.at[wid], dst_v)
        plsc.subcore_barrier()

        for k in range(DEPTH):
            pltpu.async_copy(g_sp.at[src_v.at[k]], bufs.at[k], sems[k])

        def body(t, _):
            for k in range(DEPTH):
                j = DEPTH * t + k
                pltpu.make_async_copy(
                    g_sp.at[src_v.at[j]], bufs.at[k], sems[k]
                ).wait()
                pltpu.sync_copy(bufs.at[k], s_sp.at[dst_v.at[j]], add=True)

                @pl.when(j + DEPTH < nchunk)
                def _():
                    pltpu.async_copy(
                        g_sp.at[src_v.at[j + DEPTH]], bufs.at[k], sems[k]
                    )

            return 0

        lax.fori_loop(0, nchunk // DEPTH, body, 0)
        plsc.subcore_barrier()

        sl = pl.ds(sid * out_rows, out_rows)
        pltpu.sync_copy(s_sp.at[sl], bounce_v)
        pltpu.sync_copy(
            bounce_v, s_out.at[pl.ds(cid * n + sid * out_rows, out_rows)]
        )

    return agg_kernel


def _matmul_stage(x, wcat, n, dc, block):
    def body(x_ref, w_ref, h_ref):
        h_ref[...] = jnp.dot(
            x_ref[...], w_ref[...], preferred_element_type=jnp.float32
        )

    return pl.pallas_call(
        body,
        grid=(n // block,),
        in_specs=[
            pl.BlockSpec((block, x.shape[1]), lambda i: (i, 0)),
            pl.BlockSpec((wcat.shape[0], dc), lambda i: (0, 0)),
        ],
        out_specs=pl.BlockSpec((block, dc), lambda i: (i, 0)),
        out_shape=jax.ShapeDtypeStruct((n, dc), jnp.float32),
    )(x, wcat)


def _scale_stage(h, deg, n, dc, npad):
    def body(h_ref, d0_ref, d1_ref, g_ref):
        m = (lax.broadcasted_iota(jnp.int32, (n, 1), 0) == 0).astype(jnp.float32)
        degsum = d0_ref[:, 0] + d1_ref[:, 0] + 1.0 - m[:, 0] * npad
        g_ref[...] = h_ref[...] * lax.rsqrt(degsum)[:, None]

    return pl.pallas_call(
        body,
        grid=(1,),
        in_specs=[
            pl.BlockSpec((n, dc), lambda i: (0, 0)),
            pl.BlockSpec((n, DEGW), lambda i: (0, 0)),
            pl.BlockSpec((n, DEGW), lambda i: (1, 0)),
        ],
        out_specs=pl.BlockSpec((n, dc), lambda i: (0, 0)),
        out_shape=jax.ShapeDtypeStruct((n, dc), jnp.float32),
    )(h, deg, deg)


def _finalize_stage(s_parts, g, deg, b_mu, b_logstd, n, dc, dout, npad):
    def body(s0_ref, s1_ref, g_ref, d0_ref, d1_ref, bm_ref, bl_ref,
             mu_ref, lo_ref):
        m = (lax.broadcasted_iota(jnp.int32, (n, 1), 0) == 0).astype(jnp.float32)
        degsum = d0_ref[:, 0] + d1_ref[:, 0] + 1.0 - m[:, 0] * npad
        dinv = lax.rsqrt(degsum)
        gv = g_ref[...]
        out = (s0_ref[...] + s1_ref[...] + gv - m * (npad * gv[0:1, :])) * dinv[:, None]
        mu_ref[...] = out[:, :dout] + bm_ref[...]
        lo_ref[...] = out[:, dout:] + bl_ref[...]

    return pl.pallas_call(
        body,
        grid=(1,),
        in_specs=[
            pl.BlockSpec((n, dc), lambda i: (0, 0)),
            pl.BlockSpec((n, dc), lambda i: (1, 0)),
            pl.BlockSpec((n, dc), lambda i: (0, 0)),
            pl.BlockSpec((n, DEGW), lambda i: (0, 0)),
            pl.BlockSpec((n, DEGW), lambda i: (1, 0)),
            pl.BlockSpec((1, dout), lambda i: (0, 0)),
            pl.BlockSpec((1, dout), lambda i: (0, 0)),
        ],
        out_specs=[
            pl.BlockSpec((n, dout), lambda i: (0, 0)),
            pl.BlockSpec((n, dout), lambda i: (0, 0)),
        ],
        out_shape=[
            jax.ShapeDtypeStruct((n, dout), jnp.float32),
            jax.ShapeDtypeStruct((n, dout), jnp.float32),
        ],
    )(s_parts, s_parts, g, deg, deg, b_mu, b_logstd)


@jax.jit
def kernel(x, W_mu, b_mu, W_logstd, b_logstd, edge_index):
    n, din = x.shape
    dout = W_mu.shape[1]
    dc = 2 * dout
    e = edge_index.shape[1]
    block = 1000

    e_pad = ((e + NW * CHUNK - 1) // (NW * CHUNK)) * (NW * CHUNK)
    nchunk = e_pad // (NW * CHUNK)

    npad = e_pad - e
    epad = jnp.pad(edge_index.astype(jnp.int32), ((0, 0), (0, npad)))
    src = epad[0].reshape(NW, nchunk, CHUNK)
    dst = epad[1].reshape(NW, nchunk, CHUNK)

    wcat = jnp.concatenate([W_mu, W_logstd], axis=1)

    oz = jnp.stack(
        [jnp.ones((CHUNK, DEGW), jnp.float32), jnp.zeros((CHUNK, DEGW), jnp.float32)]
    )
    h = _matmul_stage(x, wcat, n, dc, block)
    deg = _make_deg_kernel(n, nchunk)(dst, oz)
    g = _scale_stage(h, deg, n, dc, npad)
    s_parts = _make_agg_kernel(n, nchunk, dc)(src, dst, g)
    return _finalize_stage(
        s_parts, g, deg, b_mu.reshape(1, dout), b_logstd.reshape(1, dout),
        n, dc, dout, npad
    )
